# baseline (speedup 1.0000x reference)
import jax
import jax.numpy as jnp
from jax import lax
from jax.experimental import pallas as pl

_B, _S, _D = 1, 2048, 768
_H, _DK, _R = 12, 64, 16
_BW, _NB = 4.0, 64
_P = 8
_NBITS = 6
_KX = 8
_FULL = _DK + _NBITS + 1

_BP = 512
_BQ = 512
_BS = 1024


def _prep_body(wq_u, wq_v, uq_u, uq_v, wk_u, wk_v, uk_u, uk_v, wv_u, wv_v,
               wq_ref, wk_ref, wv_ref):
    f32 = jnp.float32
    for h in range(_H):
        a = jnp.dot(wq_v[h], uq_u[h], preferred_element_type=f32)
        b = jnp.dot(a, uq_v[h], preferred_element_type=f32)
        wq_ref[:, h * _DK:(h + 1) * _DK] = jnp.dot(
            wq_u[h], b, preferred_element_type=f32)
        a = jnp.dot(wk_v[h], uk_u[h], preferred_element_type=f32)
        b = jnp.dot(a, uk_v[h], preferred_element_type=f32)
        wk_ref[:, h * _DK:(h + 1) * _DK] = jnp.dot(
            wk_u[h], b, preferred_element_type=f32)
        wv_ref[:, h * _DK:(h + 1) * _DK] = jnp.dot(
            wv_u[h], wv_v[h], preferred_element_type=f32)


def _proj_body(q_ref, k_ref, v_ref, wq_ref, wk_ref, wv_ref, lsh_ref,
               qu_ref, ku_ref, vu_ref, qs_ref, ks_ref, qx_ref, kx_ref):
    f32 = jnp.float32
    i8 = jnp.int8
    qa = jnp.dot(q_ref[0], wq_ref[...], preferred_element_type=f32)
    ka = jnp.dot(k_ref[0], wk_ref[...], preferred_element_type=f32)
    va = jnp.dot(v_ref[0], wv_ref[...], preferred_element_type=f32)

    lane = lax.broadcasted_iota(jnp.int32, (_D, _H), 0)
    head = lax.broadcasted_iota(jnp.int32, (_D, _H), 1)
    in_head = (lane // _DK) == head
    bd = in_head.astype(f32)
    sel8 = (in_head & ((lane % _DK) < _P)).astype(f32)

    qmul = qa * lsh_ref[...]
    kmul = ka * lsh_ref[...]
    qbkt = jnp.mod(jnp.floor(
        jnp.dot(qmul, bd, preferred_element_type=f32) * (1.0 / _BW)),
        float(_NB))
    kbkt = jnp.mod(jnp.floor(
        jnp.dot(kmul, bd, preferred_element_type=f32) * (1.0 / _BW)),
        float(_NB))

    agree = ((qa > 0) == (ka > 0)).astype(f32)
    ins = jnp.dot(agree, sel8, preferred_element_type=f32) > (_P - 0.5)

    qsg = jnp.where(qa > 0, 1, -1).astype(i8)
    ksg = jnp.where(ka > 0, 1, -1).astype(i8)

    shifts = lax.broadcasted_iota(jnp.int32, (1, _NBITS), 1)
    rows = qa.shape[0]
    zpad = jnp.zeros((rows, _KX - _NBITS - 1), f32)
    onescol = jnp.ones((rows, 1), f32)
    for h in range(_H):
        sl = slice(h * _DK, (h + 1) * _DK)
        qu_ref[h] = qa[:, sl]
        ku_ref[h] = ka[:, sl]
        vu_ref[h] = va[:, sl]
        qs_ref[h] = qsg[:, sl]
        ks_ref[h] = ksg[:, sl]
        qbits = jnp.bitwise_and(jnp.right_shift(
            qbkt[:, h:h + 1].astype(jnp.int32), shifts), 1).astype(f32)
        kbits = jnp.bitwise_and(jnp.right_shift(
            kbkt[:, h:h + 1].astype(jnp.int32), shifts), 1).astype(f32)
        qx_ref[h] = jnp.concatenate(
            [2.0 * qbits - 1.0, onescol, zpad], axis=1).astype(i8)
        kx_ref[h] = jnp.concatenate(
            [2.0 * kbits - 1.0, ins[:, h:h + 1].astype(f32), zpad],
            axis=1).astype(i8)


def _attn_body(qs_ref, ks_ref, qx_ref, kx_ref, qb_ref, kf_ref, vf_ref,
               out_ref):
    f32 = jnp.float32
    score = (jnp.dot(qs_ref[0], ks_ref[0].T, preferred_element_type=jnp.int32)
             + jnp.dot(qx_ref[0], kx_ref[0].T,
                       preferred_element_type=jnp.int32))
    got = jnp.max(score) > (_FULL - 1)

    @pl.when(got)
    def _slow():
        qb = qb_ref[0]
        kf = kf_ref[0]
        sim = jnp.dot(qb, kf.T, preferred_element_type=f32)
        mask = (score > (_FULL - 1)) & (sim > -1e8)
        s = jnp.maximum(sim * 0.125, -1e4)
        m = jnp.max(jnp.where(mask, s, -1e30), axis=1, keepdims=True)
        e = jnp.where(mask, jnp.exp(s - m), 0.0)
        den = jnp.maximum(jnp.sum(e, axis=1, keepdims=True), 1e-9)
        p = e / den
        out_ref[0] = jnp.dot(p, vf_ref[0], preferred_element_type=f32)

    @pl.when(jnp.logical_not(got))
    def _fast():
        out_ref[0] = jnp.zeros((_BQ, _DK), f32)


def _oproj_body(ho_ref, wo_ref, out_ref):
    h = pl.program_id(1)
    ho = ho_ref[0]

    @pl.when(h == 0)
    def _init():
        out_ref[...] = jnp.zeros(out_ref.shape, jnp.float32)

    nz = jnp.any(ho != 0.0)

    @pl.when(nz)
    def _acc():
        out_ref[...] += jnp.dot(ho, wo_ref[...],
                                preferred_element_type=jnp.float32)


def kernel(query, key, value, Wq_u, Wq_v, Uq_u, Uq_v, Wk_u, Wk_v, Uk_u, Uk_v,
           Wv_u, Wv_v, lsh_vecs, Wo):
    f32 = jnp.float32
    i8 = jnp.int8

    def full(*dims):
        return pl.BlockSpec(dims, lambda: tuple(0 for _ in dims))

    wq_eff, wk_eff, wv_eff = pl.pallas_call(
        _prep_body,
        grid=(),
        in_specs=[
            full(_H, _D, _R), full(_H, _R, _DK), full(_H, _DK, _R),
            full(_H, _R, _DK), full(_H, _D, _R), full(_H, _R, _DK),
            full(_H, _DK, _R), full(_H, _R, _DK), full(_H, _D, _R),
            full(_H, _R, _DK),
        ],
        out_specs=[full(_D, _D)] * 3,
        out_shape=[jax.ShapeDtypeStruct((_D, _D), f32)] * 3,
    )(Wq_u, Wq_v, Uq_u, Uq_v, Wk_u, Wk_v, Uk_u, Uk_v, Wv_u, Wv_v)

    lsh_flat = lsh_vecs.reshape(1, _H * _DK)

    n_pb = _S // _BP
    q_up, k_up, v_up, qs, ks, qx, kx = pl.pallas_call(
        _proj_body,
        grid=(n_pb,),
        in_specs=[
            pl.BlockSpec((1, _BP, _D), lambda i: (0, i, 0)),
            pl.BlockSpec((1, _BP, _D), lambda i: (0, i, 0)),
            pl.BlockSpec((1, _BP, _D), lambda i: (0, i, 0)),
            pl.BlockSpec((_D, _D), lambda i: (0, 0)),
            pl.BlockSpec((_D, _D), lambda i: (0, 0)),
            pl.BlockSpec((_D, _D), lambda i: (0, 0)),
            pl.BlockSpec((1, _D), lambda i: (0, 0)),
        ],
        out_specs=[pl.BlockSpec((_H, _BP, _DK), lambda i: (0, i, 0))] * 5 +
                  [pl.BlockSpec((_H, _BP, _KX), lambda i: (0, i, 0))] * 2,
        out_shape=[jax.ShapeDtypeStruct((_H, _S, _DK), f32)] * 3 +
                  [jax.ShapeDtypeStruct((_H, _S, _DK), i8)] * 2 +
                  [jax.ShapeDtypeStruct((_H, _S, _KX), i8)] * 2,
    )(query, key, value, wq_eff, wk_eff, wv_eff, lsh_flat)

    n_qb = _S // _BQ
    head_out = pl.pallas_call(
        _attn_body,
        grid=(_H, n_qb),
        in_specs=[
            pl.BlockSpec((1, _BQ, _DK), lambda h, i: (h, i, 0)),
            pl.BlockSpec((1, _S, _DK), lambda h, i: (h, 0, 0)),
            pl.BlockSpec((1, _BQ, _KX), lambda h, i: (h, i, 0)),
            pl.BlockSpec((1, _S, _KX), lambda h, i: (h, 0, 0)),
            pl.BlockSpec((1, _BQ, _DK), lambda h, i: (h, i, 0)),
            pl.BlockSpec((1, _S, _DK), lambda h, i: (h, 0, 0)),
            pl.BlockSpec((1, _S, _DK), lambda h, i: (h, 0, 0)),
        ],
        out_specs=pl.BlockSpec((1, _BQ, _DK), lambda h, i: (h, i, 0)),
        out_shape=jax.ShapeDtypeStruct((_H, _S, _DK), f32),
    )(qs, ks, qx, kx, q_up, k_up, v_up)

    n_rb = _S // _BS
    out = pl.pallas_call(
        _oproj_body,
        grid=(n_rb, _H),
        in_specs=[
            pl.BlockSpec((1, _BS, _DK), lambda i, h: (h, i, 0)),
            pl.BlockSpec((_DK, _D), lambda i, h: (h, 0)),
        ],
        out_specs=pl.BlockSpec((_BS, _D), lambda i, h: (i, 0)),
        out_shape=jax.ShapeDtypeStruct((_S, _D), f32),
    )(head_out, Wo)

    return out.reshape(_B, _S, _D)


# speedup vs baseline: 1.2320x; 1.2320x over previous
import jax
import jax.numpy as jnp
from jax import lax
from jax.experimental import pallas as pl

_B, _S, _D = 1, 2048, 768
_H, _DK, _R = 12, 64, 16
_BW, _NB = 4.0, 64
_P = 8
_NBITS = 6
_KX = 8
_FULL = _DK + _NBITS + 1

_BP = 512
_BQ = 512
_BS = 1024


def _prep_body(wq_u, wq_v, uq_u, uq_v, wk_u, wk_v, uk_u, uk_v, wv_u, wv_v,
               wq_ref, wk_ref, wv_ref):
    f32 = jnp.float32
    for h in range(_H):
        a = jnp.dot(wq_v[h], uq_u[h], preferred_element_type=f32)
        b = jnp.dot(a, uq_v[h], preferred_element_type=f32)
        wq_ref[:, h * _DK:(h + 1) * _DK] = jnp.dot(
            wq_u[h], b, preferred_element_type=f32)
        a = jnp.dot(wk_v[h], uk_u[h], preferred_element_type=f32)
        b = jnp.dot(a, uk_v[h], preferred_element_type=f32)
        wk_ref[:, h * _DK:(h + 1) * _DK] = jnp.dot(
            wk_u[h], b, preferred_element_type=f32)
        wv_ref[:, h * _DK:(h + 1) * _DK] = jnp.dot(
            wv_u[h], wv_v[h], preferred_element_type=f32)


def _proj_body(q_ref, k_ref, v_ref, wq_ref, wk_ref, wv_ref, lsh_ref,
               qu_ref, ku_ref, vu_ref, qsig_ref, ksig_ref):
    f32 = jnp.float32
    i8 = jnp.int8
    bf16 = jnp.bfloat16
    qa = jnp.dot(q_ref[0].astype(bf16), wq_ref[...].astype(bf16),
                 preferred_element_type=f32)
    ka = jnp.dot(k_ref[0].astype(bf16), wk_ref[...].astype(bf16),
                 preferred_element_type=f32)
    va = jnp.dot(v_ref[0].astype(bf16), wv_ref[...].astype(bf16),
                 preferred_element_type=f32)

    lane = lax.broadcasted_iota(jnp.int32, (_D, _H), 0)
    head = lax.broadcasted_iota(jnp.int32, (_D, _H), 1)
    in_head = (lane // _DK) == head
    bd = in_head.astype(f32)
    sel8 = (in_head & ((lane % _DK) < _P)).astype(f32)
    rep = bd.T

    qmul = qa * lsh_ref[...]
    kmul = ka * lsh_ref[...]
    qbkt = jnp.mod(jnp.floor(
        jnp.dot(qmul, bd, preferred_element_type=f32) * (1.0 / _BW)),
        float(_NB))
    kbkt = jnp.mod(jnp.floor(
        jnp.dot(kmul, bd, preferred_element_type=f32) * (1.0 / _BW)),
        float(_NB))

    agree = ((qa > 0) == (ka > 0)).astype(f32)
    ins = (jnp.dot(agree, sel8, preferred_element_type=f32)
           > (_P - 0.5)).astype(f32)

    qb_rep = jnp.dot(qbkt, rep, preferred_element_type=f32)
    kb_rep = jnp.dot(kbkt, rep, preferred_element_type=f32)
    ins_rep = jnp.dot(ins, rep, preferred_element_type=f32)
    lane64 = (lane % _DK).T[0:1, :]

    qsg = jnp.where(qa > 0, 1, -1).astype(i8)
    ksg = (jnp.where(ka > 0, 1.0, -1.0) *
           (ins_rep > 0.5).astype(f32)).astype(i8)
    qoh = (qb_rep == lane64).astype(i8)
    koh = (kb_rep == lane64).astype(i8)

    for h in range(_H):
        sl = slice(h * _DK, (h + 1) * _DK)
        qu_ref[h] = qa[:, sl]
        ku_ref[h] = ka[:, sl]
        vu_ref[h] = va[:, sl]
        qsig_ref[h] = jnp.concatenate([qsg[:, sl], qoh[:, sl]], axis=1)
        ksig_ref[h] = jnp.concatenate([ksg[:, sl], koh[:, sl]], axis=1)


def _attn_body(qsig_ref, ksig_ref, qb_ref, kf_ref, vf_ref, out_ref):
    f32 = jnp.float32
    score = jnp.dot(qsig_ref[0], ksig_ref[0].T,
                    preferred_element_type=jnp.int32)
    got = jnp.max(score) > _DK

    @pl.when(got)
    def _slow():
        qb = qb_ref[0]
        kf = kf_ref[0]
        sim = jnp.dot(qb, kf.T, preferred_element_type=f32)
        mask = (score > _DK) & (sim > -1e8)
        s = jnp.maximum(sim * 0.125, -1e4)
        m = jnp.max(jnp.where(mask, s, -1e30), axis=1, keepdims=True)
        e = jnp.where(mask, jnp.exp(s - m), 0.0)
        den = jnp.maximum(jnp.sum(e, axis=1, keepdims=True), 1e-9)
        p = e / den
        out_ref[0] = jnp.dot(p, vf_ref[0], preferred_element_type=f32)

    @pl.when(jnp.logical_not(got))
    def _fast():
        out_ref[0] = jnp.zeros((_BQ, _DK), f32)


def _oproj_body(ho_ref, wo_ref, out_ref):
    h = pl.program_id(1)
    ho = ho_ref[0]

    @pl.when(h == 0)
    def _init():
        out_ref[...] = jnp.zeros(out_ref.shape, jnp.float32)

    nz = jnp.any(ho != 0.0)

    @pl.when(nz)
    def _acc():
        out_ref[...] += jnp.dot(ho, wo_ref[...],
                                preferred_element_type=jnp.float32)


def kernel(query, key, value, Wq_u, Wq_v, Uq_u, Uq_v, Wk_u, Wk_v, Uk_u, Uk_v,
           Wv_u, Wv_v, lsh_vecs, Wo):
    f32 = jnp.float32
    i8 = jnp.int8

    def full(*dims):
        return pl.BlockSpec(dims, lambda: tuple(0 for _ in dims))

    wq_eff, wk_eff, wv_eff = pl.pallas_call(
        _prep_body,
        grid=(),
        in_specs=[
            full(_H, _D, _R), full(_H, _R, _DK), full(_H, _DK, _R),
            full(_H, _R, _DK), full(_H, _D, _R), full(_H, _R, _DK),
            full(_H, _DK, _R), full(_H, _R, _DK), full(_H, _D, _R),
            full(_H, _R, _DK),
        ],
        out_specs=[full(_D, _D)] * 3,
        out_shape=[jax.ShapeDtypeStruct((_D, _D), f32)] * 3,
    )(Wq_u, Wq_v, Uq_u, Uq_v, Wk_u, Wk_v, Uk_u, Uk_v, Wv_u, Wv_v)

    lsh_flat = lsh_vecs.reshape(1, _H * _DK)

    n_pb = _S // _BP
    q_up, k_up, v_up, qsig, ksig = pl.pallas_call(
        _proj_body,
        grid=(n_pb,),
        in_specs=[
            pl.BlockSpec((1, _BP, _D), lambda i: (0, i, 0)),
            pl.BlockSpec((1, _BP, _D), lambda i: (0, i, 0)),
            pl.BlockSpec((1, _BP, _D), lambda i: (0, i, 0)),
            pl.BlockSpec((_D, _D), lambda i: (0, 0)),
            pl.BlockSpec((_D, _D), lambda i: (0, 0)),
            pl.BlockSpec((_D, _D), lambda i: (0, 0)),
            pl.BlockSpec((1, _D), lambda i: (0, 0)),
        ],
        out_specs=[pl.BlockSpec((_H, _BP, _DK), lambda i: (0, i, 0))] * 3 +
                  [pl.BlockSpec((_H, _BP, 2 * _DK), lambda i: (0, i, 0))] * 2,
        out_shape=[jax.ShapeDtypeStruct((_H, _S, _DK), f32)] * 3 +
                  [jax.ShapeDtypeStruct((_H, _S, 2 * _DK), i8)] * 2,
    )(query, key, value, wq_eff, wk_eff, wv_eff, lsh_flat)

    n_qb = _S // _BQ
    head_out = pl.pallas_call(
        _attn_body,
        grid=(_H, n_qb),
        in_specs=[
            pl.BlockSpec((1, _BQ, 2 * _DK), lambda h, i: (h, i, 0)),
            pl.BlockSpec((1, _S, 2 * _DK), lambda h, i: (h, 0, 0)),
            pl.BlockSpec((1, _BQ, _DK), lambda h, i: (h, i, 0)),
            pl.BlockSpec((1, _S, _DK), lambda h, i: (h, 0, 0)),
            pl.BlockSpec((1, _S, _DK), lambda h, i: (h, 0, 0)),
        ],
        out_specs=pl.BlockSpec((1, _BQ, _DK), lambda h, i: (h, i, 0)),
        out_shape=jax.ShapeDtypeStruct((_H, _S, _DK), f32),
    )(qsig, ksig, q_up, k_up, v_up)

    n_rb = _S // _BS
    out = pl.pallas_call(
        _oproj_body,
        grid=(n_rb, _H),
        in_specs=[
            pl.BlockSpec((1, _BS, _DK), lambda i, h: (h, i, 0)),
            pl.BlockSpec((_DK, _D), lambda i, h: (h, 0)),
        ],
        out_specs=pl.BlockSpec((_BS, _D), lambda i, h: (i, 0)),
        out_shape=jax.ShapeDtypeStruct((_S, _D), f32),
    )(head_out, Wo)

    return out.reshape(_B, _S, _D)


# speedup vs baseline: 1.3700x; 1.1121x over previous
import jax
import jax.numpy as jnp
from jax import lax
from jax.experimental import pallas as pl

_B, _S, _D = 1, 2048, 768
_H, _DK, _R = 12, 64, 16
_BW, _NB = 4.0, 64
_P = 8
_NBITS = 6
_KX = 8
_FULL = _DK + _NBITS + 1

_BP = 512
_BQ = 1024
_BS = 1024


def _prep_body(wq_u, wq_v, uq_u, uq_v, wk_u, wk_v, uk_u, uk_v, wv_u, wv_v,
               wq_ref, wk_ref, wv_ref):
    f32 = jnp.float32
    for h in range(_H):
        a = jnp.dot(wq_v[h], uq_u[h], preferred_element_type=f32)
        b = jnp.dot(a, uq_v[h], preferred_element_type=f32)
        wq_ref[:, h * _DK:(h + 1) * _DK] = jnp.dot(
            wq_u[h], b, preferred_element_type=f32)
        a = jnp.dot(wk_v[h], uk_u[h], preferred_element_type=f32)
        b = jnp.dot(a, uk_v[h], preferred_element_type=f32)
        wk_ref[:, h * _DK:(h + 1) * _DK] = jnp.dot(
            wk_u[h], b, preferred_element_type=f32)
        wv_ref[:, h * _DK:(h + 1) * _DK] = jnp.dot(
            wv_u[h], wv_v[h], preferred_element_type=f32)


def _proj_body(q_ref, k_ref, v_ref, wq_ref, wk_ref, wv_ref, lsh_ref,
               qu_ref, ku_ref, vu_ref, qsig_ref, ksig_ref):
    f32 = jnp.float32
    i8 = jnp.int8
    bf16 = jnp.bfloat16
    qa = jnp.dot(q_ref[0].astype(bf16), wq_ref[...].astype(bf16),
                 preferred_element_type=f32)
    ka = jnp.dot(k_ref[0].astype(bf16), wk_ref[...].astype(bf16),
                 preferred_element_type=f32)
    va = jnp.dot(v_ref[0].astype(bf16), wv_ref[...].astype(bf16),
                 preferred_element_type=f32)

    lane = lax.broadcasted_iota(jnp.int32, (_D, _H), 0)
    head = lax.broadcasted_iota(jnp.int32, (_D, _H), 1)
    in_head = (lane // _DK) == head
    bd = in_head.astype(f32)
    sel8 = (in_head & ((lane % _DK) < _P)).astype(f32)
    rep = bd.T

    qmul = qa * lsh_ref[...]
    kmul = ka * lsh_ref[...]
    qbkt = jnp.mod(jnp.floor(
        jnp.dot(qmul, bd, preferred_element_type=f32) * (1.0 / _BW)),
        float(_NB))
    kbkt = jnp.mod(jnp.floor(
        jnp.dot(kmul, bd, preferred_element_type=f32) * (1.0 / _BW)),
        float(_NB))

    agree = ((qa > 0) == (ka > 0)).astype(f32)
    ins = (jnp.dot(agree, sel8, preferred_element_type=f32)
           > (_P - 0.5)).astype(f32)

    qb_rep = jnp.dot(qbkt, rep, preferred_element_type=f32)
    kb_rep = jnp.dot(kbkt, rep, preferred_element_type=f32)
    ins_rep = jnp.dot(ins, rep, preferred_element_type=f32)
    lane64 = (lane % _DK).T[0:1, :]

    qsg = jnp.where(qa > 0, 1, -1).astype(i8)
    ksg = (jnp.where(ka > 0, 1.0, -1.0) *
           (ins_rep > 0.5).astype(f32)).astype(i8)
    qoh = (qb_rep == lane64).astype(i8)
    koh = (kb_rep == lane64).astype(i8)

    for h in range(_H):
        sl = slice(h * _DK, (h + 1) * _DK)
        qu_ref[h] = qa[:, sl]
        ku_ref[h] = ka[:, sl]
        vu_ref[h] = va[:, sl]
        qsig_ref[h] = jnp.concatenate([qsg[:, sl], qoh[:, sl]], axis=1)
        ksig_ref[h] = jnp.concatenate([ksg[:, sl], koh[:, sl]], axis=1)


def _attn_body(qsig_ref, ksig_ref, qb_ref, kf_ref, vf_ref, out_ref):
    f32 = jnp.float32
    score = jnp.dot(qsig_ref[0], ksig_ref[0].T,
                    preferred_element_type=jnp.int32)
    got = jnp.max(score) > _DK

    @pl.when(got)
    def _slow():
        qb = qb_ref[0]
        kf = kf_ref[0]
        sim = jnp.dot(qb, kf.T, preferred_element_type=f32)
        mask = (score > _DK) & (sim > -1e8)
        s = jnp.maximum(sim * 0.125, -1e4)
        m = jnp.max(jnp.where(mask, s, -1e30), axis=1, keepdims=True)
        e = jnp.where(mask, jnp.exp(s - m), 0.0)
        den = jnp.maximum(jnp.sum(e, axis=1, keepdims=True), 1e-9)
        p = e / den
        out_ref[0] = jnp.dot(p, vf_ref[0], preferred_element_type=f32)

    @pl.when(jnp.logical_not(got))
    def _fast():
        out_ref[0] = jnp.zeros((_BQ, _DK), f32)


def _oproj_body(ho_ref, wo_ref, out_ref):
    h = pl.program_id(1)
    ho = ho_ref[0]

    @pl.when(h == 0)
    def _init():
        out_ref[...] = jnp.zeros(out_ref.shape, jnp.float32)

    nz = jnp.any(ho != 0.0)

    @pl.when(nz)
    def _acc():
        out_ref[...] += jnp.dot(ho, wo_ref[...],
                                preferred_element_type=jnp.float32)


def kernel(query, key, value, Wq_u, Wq_v, Uq_u, Uq_v, Wk_u, Wk_v, Uk_u, Uk_v,
           Wv_u, Wv_v, lsh_vecs, Wo):
    f32 = jnp.float32
    i8 = jnp.int8

    def full(*dims):
        return pl.BlockSpec(dims, lambda: tuple(0 for _ in dims))

    wq_eff, wk_eff, wv_eff = pl.pallas_call(
        _prep_body,
        grid=(),
        in_specs=[
            full(_H, _D, _R), full(_H, _R, _DK), full(_H, _DK, _R),
            full(_H, _R, _DK), full(_H, _D, _R), full(_H, _R, _DK),
            full(_H, _DK, _R), full(_H, _R, _DK), full(_H, _D, _R),
            full(_H, _R, _DK),
        ],
        out_specs=[full(_D, _D)] * 3,
        out_shape=[jax.ShapeDtypeStruct((_D, _D), f32)] * 3,
    )(Wq_u, Wq_v, Uq_u, Uq_v, Wk_u, Wk_v, Uk_u, Uk_v, Wv_u, Wv_v)

    lsh_flat = lsh_vecs.reshape(1, _H * _DK)

    n_pb = _S // _BP
    q_up, k_up, v_up, qsig, ksig = pl.pallas_call(
        _proj_body,
        grid=(n_pb,),
        in_specs=[
            pl.BlockSpec((1, _BP, _D), lambda i: (0, i, 0)),
            pl.BlockSpec((1, _BP, _D), lambda i: (0, i, 0)),
            pl.BlockSpec((1, _BP, _D), lambda i: (0, i, 0)),
            pl.BlockSpec((_D, _D), lambda i: (0, 0)),
            pl.BlockSpec((_D, _D), lambda i: (0, 0)),
            pl.BlockSpec((_D, _D), lambda i: (0, 0)),
            pl.BlockSpec((1, _D), lambda i: (0, 0)),
        ],
        out_specs=[pl.BlockSpec((_H, _BP, _DK), lambda i: (0, i, 0))] * 3 +
                  [pl.BlockSpec((_H, _BP, 2 * _DK), lambda i: (0, i, 0))] * 2,
        out_shape=[jax.ShapeDtypeStruct((_H, _S, _DK), f32)] * 3 +
                  [jax.ShapeDtypeStruct((_H, _S, 2 * _DK), i8)] * 2,
    )(query, key, value, wq_eff, wk_eff, wv_eff, lsh_flat)

    n_qb = _S // _BQ
    head_out = pl.pallas_call(
        _attn_body,
        grid=(_H, n_qb),
        in_specs=[
            pl.BlockSpec((1, _BQ, 2 * _DK), lambda h, i: (h, i, 0)),
            pl.BlockSpec((1, _S, 2 * _DK), lambda h, i: (h, 0, 0)),
            pl.BlockSpec((1, _BQ, _DK), lambda h, i: (h, i, 0)),
            pl.BlockSpec((1, _S, _DK), lambda h, i: (h, 0, 0)),
            pl.BlockSpec((1, _S, _DK), lambda h, i: (h, 0, 0)),
        ],
        out_specs=pl.BlockSpec((1, _BQ, _DK), lambda h, i: (h, i, 0)),
        out_shape=jax.ShapeDtypeStruct((_H, _S, _DK), f32),
    )(qsig, ksig, q_up, k_up, v_up)

    n_rb = _S // _BS
    out = pl.pallas_call(
        _oproj_body,
        grid=(n_rb, _H),
        in_specs=[
            pl.BlockSpec((1, _BS, _DK), lambda i, h: (h, i, 0)),
            pl.BlockSpec((_DK, _D), lambda i, h: (h, 0)),
        ],
        out_specs=pl.BlockSpec((_BS, _D), lambda i, h: (i, 0)),
        out_shape=jax.ShapeDtypeStruct((_S, _D), f32),
    )(head_out, Wo)

    return out.reshape(_B, _S, _D)


# speedup vs baseline: 1.4234x; 1.0390x over previous
import jax
import jax.numpy as jnp
from jax import lax
from jax.experimental import pallas as pl

_B, _S, _D = 1, 2048, 768
_H, _DK, _R = 12, 64, 16
_BW, _NB = 4.0, 64
_P = 8
_NBITS = 6
_KX = 8
_FULL = _DK + _NBITS + 1

_BP = 512
_BQ = 2048
_BS = 1024


def _prep_body(wq_u, wq_v, uq_u, uq_v, wk_u, wk_v, uk_u, uk_v, wv_u, wv_v,
               wq_ref, wk_ref, wv_ref):
    f32 = jnp.float32
    for h in range(_H):
        a = jnp.dot(wq_v[h], uq_u[h], preferred_element_type=f32)
        b = jnp.dot(a, uq_v[h], preferred_element_type=f32)
        wq_ref[:, h * _DK:(h + 1) * _DK] = jnp.dot(
            wq_u[h], b, preferred_element_type=f32)
        a = jnp.dot(wk_v[h], uk_u[h], preferred_element_type=f32)
        b = jnp.dot(a, uk_v[h], preferred_element_type=f32)
        wk_ref[:, h * _DK:(h + 1) * _DK] = jnp.dot(
            wk_u[h], b, preferred_element_type=f32)
        wv_ref[:, h * _DK:(h + 1) * _DK] = jnp.dot(
            wv_u[h], wv_v[h], preferred_element_type=f32)


def _proj_body(q_ref, k_ref, v_ref, wq_ref, wk_ref, wv_ref, lsh_ref,
               qu_ref, ku_ref, vu_ref, qsig_ref, ksig_ref):
    f32 = jnp.float32
    i8 = jnp.int8
    bf16 = jnp.bfloat16
    qa = jnp.dot(q_ref[0].astype(bf16), wq_ref[...].astype(bf16),
                 preferred_element_type=f32)
    ka = jnp.dot(k_ref[0].astype(bf16), wk_ref[...].astype(bf16),
                 preferred_element_type=f32)
    va = jnp.dot(v_ref[0].astype(bf16), wv_ref[...].astype(bf16),
                 preferred_element_type=f32)

    lane = lax.broadcasted_iota(jnp.int32, (_D, _H), 0)
    head = lax.broadcasted_iota(jnp.int32, (_D, _H), 1)
    in_head = (lane // _DK) == head
    bd = in_head.astype(f32)
    sel8 = (in_head & ((lane % _DK) < _P)).astype(f32)
    rep = bd.T

    qmul = qa * lsh_ref[...]
    kmul = ka * lsh_ref[...]
    qbkt = jnp.mod(jnp.floor(
        jnp.dot(qmul, bd, preferred_element_type=f32) * (1.0 / _BW)),
        float(_NB))
    kbkt = jnp.mod(jnp.floor(
        jnp.dot(kmul, bd, preferred_element_type=f32) * (1.0 / _BW)),
        float(_NB))

    agree = ((qa > 0) == (ka > 0)).astype(f32)
    ins = (jnp.dot(agree, sel8, preferred_element_type=f32)
           > (_P - 0.5)).astype(f32)

    qb_rep = jnp.dot(qbkt, rep, preferred_element_type=f32)
    kb_rep = jnp.dot(kbkt, rep, preferred_element_type=f32)
    ins_rep = jnp.dot(ins, rep, preferred_element_type=f32)
    lane64 = (lane % _DK).T[0:1, :]

    qsg = jnp.where(qa > 0, 1, -1).astype(i8)
    ksg = (jnp.where(ka > 0, 1.0, -1.0) *
           (ins_rep > 0.5).astype(f32)).astype(i8)
    qoh = (qb_rep == lane64).astype(i8)
    koh = (kb_rep == lane64).astype(i8)

    for h in range(_H):
        sl = slice(h * _DK, (h + 1) * _DK)
        qu_ref[h] = qa[:, sl]
        ku_ref[h] = ka[:, sl]
        vu_ref[h] = va[:, sl]
        qsig_ref[h] = jnp.concatenate([qsg[:, sl], qoh[:, sl]], axis=1)
        ksig_ref[h] = jnp.concatenate([ksg[:, sl], koh[:, sl]], axis=1)


def _attn_body(qsig_ref, ksig_ref, qb_ref, kf_ref, vf_ref, out_ref):
    f32 = jnp.float32
    score = jnp.dot(qsig_ref[0], ksig_ref[0].T,
                    preferred_element_type=jnp.int32)
    got = jnp.max(score) > _DK

    @pl.when(got)
    def _slow():
        qb = qb_ref[0]
        kf = kf_ref[0]
        sim = jnp.dot(qb, kf.T, preferred_element_type=f32)
        mask = (score > _DK) & (sim > -1e8)
        s = jnp.maximum(sim * 0.125, -1e4)
        m = jnp.max(jnp.where(mask, s, -1e30), axis=1, keepdims=True)
        e = jnp.where(mask, jnp.exp(s - m), 0.0)
        den = jnp.maximum(jnp.sum(e, axis=1, keepdims=True), 1e-9)
        p = e / den
        out_ref[0] = jnp.dot(p, vf_ref[0], preferred_element_type=f32)

    @pl.when(jnp.logical_not(got))
    def _fast():
        out_ref[0] = jnp.zeros((_BQ, _DK), f32)


def _oproj_body(ho_ref, wo_ref, out_ref):
    h = pl.program_id(1)
    ho = ho_ref[0]

    @pl.when(h == 0)
    def _init():
        out_ref[...] = jnp.zeros(out_ref.shape, jnp.float32)

    nz = jnp.any(ho != 0.0)

    @pl.when(nz)
    def _acc():
        out_ref[...] += jnp.dot(ho, wo_ref[...],
                                preferred_element_type=jnp.float32)


def kernel(query, key, value, Wq_u, Wq_v, Uq_u, Uq_v, Wk_u, Wk_v, Uk_u, Uk_v,
           Wv_u, Wv_v, lsh_vecs, Wo):
    f32 = jnp.float32
    i8 = jnp.int8

    def full(*dims):
        return pl.BlockSpec(dims, lambda: tuple(0 for _ in dims))

    wq_eff, wk_eff, wv_eff = pl.pallas_call(
        _prep_body,
        grid=(),
        in_specs=[
            full(_H, _D, _R), full(_H, _R, _DK), full(_H, _DK, _R),
            full(_H, _R, _DK), full(_H, _D, _R), full(_H, _R, _DK),
            full(_H, _DK, _R), full(_H, _R, _DK), full(_H, _D, _R),
            full(_H, _R, _DK),
        ],
        out_specs=[full(_D, _D)] * 3,
        out_shape=[jax.ShapeDtypeStruct((_D, _D), f32)] * 3,
    )(Wq_u, Wq_v, Uq_u, Uq_v, Wk_u, Wk_v, Uk_u, Uk_v, Wv_u, Wv_v)

    lsh_flat = lsh_vecs.reshape(1, _H * _DK)

    n_pb = _S // _BP
    q_up, k_up, v_up, qsig, ksig = pl.pallas_call(
        _proj_body,
        grid=(n_pb,),
        in_specs=[
            pl.BlockSpec((1, _BP, _D), lambda i: (0, i, 0)),
            pl.BlockSpec((1, _BP, _D), lambda i: (0, i, 0)),
            pl.BlockSpec((1, _BP, _D), lambda i: (0, i, 0)),
            pl.BlockSpec((_D, _D), lambda i: (0, 0)),
            pl.BlockSpec((_D, _D), lambda i: (0, 0)),
            pl.BlockSpec((_D, _D), lambda i: (0, 0)),
            pl.BlockSpec((1, _D), lambda i: (0, 0)),
        ],
        out_specs=[pl.BlockSpec((_H, _BP, _DK), lambda i: (0, i, 0))] * 3 +
                  [pl.BlockSpec((_H, _BP, 2 * _DK), lambda i: (0, i, 0))] * 2,
        out_shape=[jax.ShapeDtypeStruct((_H, _S, _DK), f32)] * 3 +
                  [jax.ShapeDtypeStruct((_H, _S, 2 * _DK), i8)] * 2,
    )(query, key, value, wq_eff, wk_eff, wv_eff, lsh_flat)

    n_qb = _S // _BQ
    head_out = pl.pallas_call(
        _attn_body,
        grid=(_H, n_qb),
        in_specs=[
            pl.BlockSpec((1, _BQ, 2 * _DK), lambda h, i: (h, i, 0)),
            pl.BlockSpec((1, _S, 2 * _DK), lambda h, i: (h, 0, 0)),
            pl.BlockSpec((1, _BQ, _DK), lambda h, i: (h, i, 0)),
            pl.BlockSpec((1, _S, _DK), lambda h, i: (h, 0, 0)),
            pl.BlockSpec((1, _S, _DK), lambda h, i: (h, 0, 0)),
        ],
        out_specs=pl.BlockSpec((1, _BQ, _DK), lambda h, i: (h, i, 0)),
        out_shape=jax.ShapeDtypeStruct((_H, _S, _DK), f32),
    )(qsig, ksig, q_up, k_up, v_up)

    n_rb = _S // _BS
    out = pl.pallas_call(
        _oproj_body,
        grid=(n_rb, _H),
        in_specs=[
            pl.BlockSpec((1, _BS, _DK), lambda i, h: (h, i, 0)),
            pl.BlockSpec((_DK, _D), lambda i, h: (h, 0)),
        ],
        out_specs=pl.BlockSpec((_BS, _D), lambda i, h: (i, 0)),
        out_shape=jax.ShapeDtypeStruct((_S, _D), f32),
    )(head_out, Wo)

    return out.reshape(_B, _S, _D)
